# baseline (speedup 1.0000x reference)
import jax
import jax.numpy as jnp
from jax.experimental import pallas as pl

N_FIELDS = 26
N_CLASSES = 1000
BR = 128


def _onehot_kernel(x_ref, o_ref):
    xb = x_ref[...]
    idx = jnp.where(xb == 0, 0, xb - 1)
    k = jax.lax.broadcasted_iota(jnp.int32, (BR, N_FIELDS, N_CLASSES), 2)
    o_ref[...] = jnp.where(k == idx[:, :, None], -1.0, 0.0)


def kernel(x):
    n = x.shape[0]
    out = pl.pallas_call(
        _onehot_kernel,
        grid=(n // BR,),
        in_specs=[pl.BlockSpec((BR, N_FIELDS), lambda i: (i, 0))],
        out_specs=pl.BlockSpec((BR, N_FIELDS, N_CLASSES), lambda i: (i, 0, 0)),
        out_shape=jax.ShapeDtypeStruct((n, N_FIELDS, N_CLASSES), jnp.float32),
    )(x)
    return out.reshape(n, N_FIELDS * N_CLASSES)
